# baseline (speedup 1.0000x reference)
import jax
import jax.numpy as jnp
from jax.experimental import pallas as pl
from jax.experimental.pallas import tpu as pltpu


def _support_body(x_ref, w1_ref, s1_ref):
    s1_ref[...] = jnp.dot(
        x_ref[...].astype(jnp.bfloat16),
        w1_ref[...].astype(jnp.bfloat16),
        preferred_element_type=jnp.float32,
    )


def _layer1_body(adj_ref, s1_ref, b1_ref, w2_ref, s2_ref):
    h = jnp.dot(
        adj_ref[...].astype(jnp.bfloat16),
        s1_ref[...].astype(jnp.bfloat16),
        preferred_element_type=jnp.float32,
    ) + b1_ref[...]
    s2_ref[...] = jnp.dot(
        h.astype(jnp.bfloat16),
        w2_ref[...].astype(jnp.bfloat16),
        preferred_element_type=jnp.float32,
    )


def _layer2_body(adj_ref, s2_ref, b2_ref, out_ref):
    out_ref[...] = jnp.dot(
        adj_ref[...].astype(jnp.bfloat16),
        s2_ref[...].astype(jnp.bfloat16),
        preferred_element_type=jnp.float32,
    ) + b2_ref[...]


def kernel(x, adj, W1, b1, W2, b2):
    N, d_in = x.shape
    d_hid = W1.shape[1]
    d_out = W2.shape[1]
    b1r = b1.reshape(1, d_hid)
    b2r = b2.reshape(1, d_out)

    bx = 1024
    s1 = pl.pallas_call(
        _support_body,
        grid=(pl.cdiv(N, bx),),
        in_specs=[
            pl.BlockSpec((bx, d_in), lambda i: (i, 0)),
            pl.BlockSpec((d_in, d_hid), lambda i: (0, 0)),
        ],
        out_specs=pl.BlockSpec((bx, d_hid), lambda i: (i, 0)),
        out_shape=jax.ShapeDtypeStruct((N, d_hid), jnp.float32),
        compiler_params=pltpu.CompilerParams(
            dimension_semantics=("parallel",),
        ),
    )(x, W1)

    bm = 512
    grid = (pl.cdiv(N, bm),)
    s2 = pl.pallas_call(
        _layer1_body,
        grid=grid,
        in_specs=[
            pl.BlockSpec((bm, N), lambda i: (i, 0)),
            pl.BlockSpec((N, d_hid), lambda i: (0, 0)),
            pl.BlockSpec((1, d_hid), lambda i: (0, 0)),
            pl.BlockSpec((d_hid, d_out), lambda i: (0, 0)),
        ],
        out_specs=pl.BlockSpec((bm, d_out), lambda i: (i, 0)),
        out_shape=jax.ShapeDtypeStruct((N, d_out), jnp.float32),
        compiler_params=pltpu.CompilerParams(
            dimension_semantics=("parallel",),
        ),
    )(adj, s1, b1r, W2)

    out = pl.pallas_call(
        _layer2_body,
        grid=grid,
        in_specs=[
            pl.BlockSpec((bm, N), lambda i: (i, 0)),
            pl.BlockSpec((N, d_out), lambda i: (0, 0)),
            pl.BlockSpec((1, d_out), lambda i: (0, 0)),
        ],
        out_specs=pl.BlockSpec((bm, d_out), lambda i: (i, 0)),
        out_shape=jax.ShapeDtypeStruct((N, d_out), jnp.float32),
        compiler_params=pltpu.CompilerParams(
            dimension_semantics=("parallel",),
        ),
    )(adj, s2, b2r)
    return out
